# baseline (speedup 1.0000x reference)
import functools
import math

import jax
import jax.numpy as jnp
from jax.experimental import pallas as pl
from jax.experimental.pallas import tpu as pltpu

_SCALE = 64.0
_MARGIN = 0.5
_COS_M = math.cos(_MARGIN)
_SIN_M = math.sin(_MARGIN)

_R = 8
_CW = 1024


def _loss_body(lab_ref, x_ref, out_ref, *, n_rows, n_cols):
    i = pl.program_id(0)
    lab = lab_ref[...]

    n_full = n_cols // _CW
    tail = n_cols - n_full * _CW
    iota = jax.lax.broadcasted_iota(jnp.int32, (_R, _CW), 1)

    def col_body(j, carry):
        acc, accp = carry
        base = j * _CW
        chunk = x_ref[:, pl.ds(base, _CW)]
        xc = jnp.clip(chunk, -1.0, 1.0)
        acc = acc + jnp.exp(xc * _SCALE - _SCALE)
        accp = accp + jnp.where(base + iota == lab, xc, 0.0)
        return acc, accp

    init = (jnp.zeros((_R, _CW), jnp.float32), jnp.zeros((_R, _CW), jnp.float32))
    acc, accp = jax.lax.fori_loop(0, n_full, col_body, init)

    s0 = jnp.sum(acc, axis=1)
    c = jnp.sum(accp, axis=1)
    if tail:
        chunk = x_ref[:, pl.ds(n_full * _CW, tail)]
        xc = jnp.clip(chunk, -1.0, 1.0)
        s0 = s0 + jnp.sum(jnp.exp(xc * _SCALE - _SCALE), axis=1)
        cols = n_full * _CW + jax.lax.broadcasted_iota(jnp.int32, (_R, tail), 1)
        c = c + jnp.sum(jnp.where(cols == lab, xc, 0.0), axis=1)

    t_new = _SCALE * (c * _COS_M - _SIN_M * jnp.sqrt(jnp.maximum(1.0 - c * c, 0.0)))
    e_old = jnp.exp(c * _SCALE - _SCALE)
    e_new = jnp.exp(t_new - _SCALE)
    s = s0 - e_old + e_new
    row_loss = _SCALE + jnp.log(s) - t_new

    @pl.when(i == 0)
    def _():
        out_ref[0, 0] = 0.0

    out_ref[0, 0] += jnp.sum(row_loss) * (1.0 / n_rows)


@jax.jit
def kernel(logits, labels):
    n_rows, n_cols = logits.shape
    lab2 = labels.reshape(n_rows, 1).astype(jnp.int32)
    out = pl.pallas_call(
        functools.partial(_loss_body, n_rows=n_rows, n_cols=n_cols),
        grid=(n_rows // _R,),
        in_specs=[
            pl.BlockSpec((_R, 1), lambda i: (i, 0)),
            pl.BlockSpec((_R, n_cols), lambda i: (i, 0)),
        ],
        out_specs=pl.BlockSpec((1, 1), lambda i: (0, 0), memory_space=pltpu.SMEM),
        out_shape=jax.ShapeDtypeStruct((1, 1), jnp.float32),
    )(lab2, logits)
    return out[0, 0]


# speedup vs baseline: 1.4634x; 1.4634x over previous
import functools
import math

import jax
import jax.numpy as jnp
from jax.experimental import pallas as pl
from jax.experimental.pallas import tpu as pltpu

_SCALE = 64.0
_MARGIN = 0.5
_COS_M = math.cos(_MARGIN)
_SIN_M = math.sin(_MARGIN)

_R = 8
_CW = 1024


def _loss_body(lab_ref, x_ref, out_ref, *, n_rows, n_cols):
    i = pl.program_id(0)
    lab = lab_ref[...]

    n_full = n_cols // _CW
    tail = n_cols - n_full * _CW
    iota = jax.lax.broadcasted_iota(jnp.int32, (_R, _CW), 1)

    def tree128(v):
        parts = [v[:, k * 128:(k + 1) * 128] for k in range(v.shape[1] // 128)]
        while len(parts) > 1:
            half = (len(parts) + 1) // 2
            parts = [
                parts[m] + parts[m + half] if m + half < len(parts) else parts[m]
                for m in range(half)
            ]
        return parts[0]

    def col_body(j, carry):
        acc, accp = carry
        base = j * _CW
        chunk = x_ref[:, pl.ds(base, _CW)]
        xc = jnp.clip(chunk, -1.0, 1.0)
        acc = acc + tree128(jnp.exp(xc * _SCALE - _SCALE))
        accp = accp + tree128(jnp.where(base + iota == lab, xc, 0.0))
        return acc, accp

    init = (jnp.zeros((_R, 128), jnp.float32), jnp.zeros((_R, 128), jnp.float32))
    acc, accp = jax.lax.fori_loop(0, n_full, col_body, init, unroll=2)

    s0 = jnp.sum(acc, axis=1)
    c = jnp.sum(accp, axis=1)
    if tail:
        chunk = x_ref[:, pl.ds(n_full * _CW, tail)]
        xc = jnp.clip(chunk, -1.0, 1.0)
        s0 = s0 + jnp.sum(jnp.exp(xc * _SCALE - _SCALE), axis=1)
        cols = n_full * _CW + jax.lax.broadcasted_iota(jnp.int32, (_R, tail), 1)
        c = c + jnp.sum(jnp.where(cols == lab, xc, 0.0), axis=1)

    t_new = _SCALE * (c * _COS_M - _SIN_M * jnp.sqrt(jnp.maximum(1.0 - c * c, 0.0)))
    e_old = jnp.exp(c * _SCALE - _SCALE)
    e_new = jnp.exp(t_new - _SCALE)
    s = s0 - e_old + e_new
    row_loss = _SCALE + jnp.log(s) - t_new

    @pl.when(i == 0)
    def _():
        out_ref[0, 0] = 0.0

    out_ref[0, 0] += jnp.sum(row_loss) * (1.0 / n_rows)


@jax.jit
def kernel(logits, labels):
    n_rows, n_cols = logits.shape
    lab2 = labels.reshape(n_rows, 1).astype(jnp.int32)
    out = pl.pallas_call(
        functools.partial(_loss_body, n_rows=n_rows, n_cols=n_cols),
        grid=(n_rows // _R,),
        in_specs=[
            pl.BlockSpec((_R, 1), lambda i: (i, 0)),
            pl.BlockSpec((_R, n_cols), lambda i: (i, 0)),
        ],
        out_specs=pl.BlockSpec((1, 1), lambda i: (0, 0), memory_space=pltpu.SMEM),
        out_shape=jax.ShapeDtypeStruct((1, 1), jnp.float32),
    )(lab2, logits)
    return out[0, 0]
